# baseline (speedup 1.0000x reference)
import jax
import jax.numpy as jnp
from jax.experimental import pallas as pl
from jax.experimental.pallas import tpu as pltpu


def _add_block(x_ref, sp_ref, tp_ref, o_ref):
    o_ref[...] = x_ref[...] + sp_ref[...] + tp_ref[...]


def kernel(x, spatial_table, temporal_table):
    batch, seqlen, height, width, d = x.shape
    hw = height * width
    x3 = x.reshape(batch * seqlen, hw, d)
    tt_tiled = jnp.tile(temporal_table[:seqlen], (batch, 1)).reshape(
        batch * seqlen, 1, d
    )

    rchunk = 8
    out = pl.pallas_call(
        _add_block,
        compiler_params=pltpu.CompilerParams(
            vmem_limit_bytes=64 * 1024 * 1024,
        ),
        grid=(batch * seqlen // rchunk,),
        in_specs=[
            pl.BlockSpec((rchunk, hw, d), lambda i: (i, 0, 0)),
            pl.BlockSpec((hw, d), lambda i: (0, 0)),
            pl.BlockSpec((rchunk, 1, d), lambda i: (i, 0, 0)),
        ],
        out_specs=pl.BlockSpec((rchunk, hw, d), lambda i: (i, 0, 0)),
        out_shape=jax.ShapeDtypeStruct((batch * seqlen, hw, d), x.dtype),
    )(x3, spatial_table, tt_tiled)

    return out.reshape(batch, seqlen, hw, d)
